# Initial kernel scaffold; baseline (speedup 1.0000x reference)
import jax
import jax.numpy as jnp
from jax.experimental import pallas as pl


def kernel(x, edge_index, W_l0, b_l0, W_r0, a0, W_l1, b_l1, W_r1, a1):
    raise NotImplementedError("write your pallas kernel here")


# speedup vs baseline: 2.9484x; 2.9484x over previous
import functools

import jax
import jax.numpy as jnp
from jax import lax
from jax.experimental import pallas as pl
from jax.experimental.pallas import tpu as pltpu
from jax.experimental.pallas import tpu_sc as plsc

_N = 10000
_E = 320000
_D = 128
_NC = 2
_NS = 16
_NW = _NC * _NS
_L = 16
_NPAD = 10112
_RPT = _NPAD // _NS
_EPAD = 327680
_EPT = _EPAD // _NW
_CH = 128
_NCHUNK = _EPT // _CH
_BLK = 2000

_mesh = functools.lru_cache(maxsize=None)(
    lambda: plsc.VectorSubcoreMesh(core_axis_name="c", subcore_axis_name="s",
                                   num_cores=_NC, num_subcores=_NS))


def _fill(ref, value):
  v16 = jnp.full((_L,), value, jnp.float32)

  def row(r, carry):
    for cc in range(_D // _L):
      ref[r, pl.ds(cc * _L, _L)] = v16
    return carry

  lax.fori_loop(0, _CH, row, 0)


def _clear_stripe(acc, rbase, zeros_v):
  for k in range(4):
    pltpu.sync_copy(zeros_v, acc.at[pl.ds(rbase + k * _CH, _CH)])
  pltpu.sync_copy(zeros_v.at[pl.ds(0, _RPT - 4 * _CH)],
                  acc.at[pl.ds(rbase + 4 * _CH, _RPT - 4 * _CH)])


@functools.lru_cache(maxsize=None)
def _make_agg():
  def body(y_hbm, src_hbm, dst_hbm, part_hbm, acc, src_v, dst_v, rows_v, sem):
    c = lax.axis_index("c")
    s = lax.axis_index("s")
    wid = s * _NC + c
    ebase = wid * _EPT
    rbase = s * _RPT

    _fill(rows_v, 0.0)
    _clear_stripe(acc, rbase, rows_v)
    plsc.subcore_barrier()

    def chunk(j, carry):
      off = ebase + j * _CH
      pltpu.sync_copy(src_hbm.at[pl.ds(off, _CH)], src_v)
      pltpu.sync_copy(dst_hbm.at[pl.ds(off, _CH)], dst_v)
      pltpu.async_copy(y_hbm.at[src_v], rows_v, sem).wait()
      pltpu.sync_copy(rows_v, acc.at[dst_v], add=True)
      return carry

    lax.fori_loop(0, _NCHUNK, chunk, 0)
    plsc.subcore_barrier()

    pltpu.sync_copy(acc.at[pl.ds(rbase, _RPT)],
                    part_hbm.at[c, pl.ds(rbase, _RPT)])

  return pl.kernel(
      body,
      out_type=jax.ShapeDtypeStruct((_NC, _NPAD, _D), jnp.float32),
      mesh=_mesh(),
      scratch_types=[
          pltpu.VMEM_SHARED((_NPAD, _D), jnp.float32),
          pltpu.VMEM((_CH,), jnp.int32),
          pltpu.VMEM((_CH,), jnp.int32),
          pltpu.VMEM((_CH, _D), jnp.float32),
          pltpu.SemaphoreType.DMA,
      ])


@functools.lru_cache(maxsize=None)
def _make_cnt():
  def body(dst_hbm, cnt_hbm, acc, dst_v, ones_v):
    c = lax.axis_index("c")
    s = lax.axis_index("s")
    wid = s * _NC + c
    ebase = wid * _EPT
    rbase = s * _RPT

    _fill(ones_v, 0.0)
    _clear_stripe(acc, rbase, ones_v)
    _fill(ones_v, 1.0)
    plsc.subcore_barrier()

    def chunk(j, carry):
      off = ebase + j * _CH
      pltpu.sync_copy(dst_hbm.at[pl.ds(off, _CH)], dst_v)
      pltpu.sync_copy(ones_v, acc.at[dst_v], add=True)
      return carry

    lax.fori_loop(0, _NCHUNK, chunk, 0)
    plsc.subcore_barrier()

    pltpu.sync_copy(acc.at[pl.ds(rbase, _RPT)],
                    cnt_hbm.at[c, pl.ds(rbase, _RPT)])

  return pl.kernel(
      body,
      out_type=jax.ShapeDtypeStruct((_NC, _NPAD, _D), jnp.float32),
      mesh=_mesh(),
      scratch_types=[
          pltpu.VMEM_SHARED((_NPAD, _D), jnp.float32),
          pltpu.VMEM((_CH,), jnp.int32),
          pltpu.VMEM((_CH, _D), jnp.float32),
      ])


def _pre_body(h_ref, wl_ref, wr_ref, b_ref, y_ref, r_ref):
  h = h_ref[...]
  y_ref[...] = jnp.dot(h, wl_ref[...], preferred_element_type=jnp.float32)
  r_ref[...] = jnp.dot(h, wr_ref[...], preferred_element_type=jnp.float32) + b_ref[...]


_pre = pl.pallas_call(
    _pre_body,
    grid=(_N // _BLK,),
    in_specs=[
        pl.BlockSpec((_BLK, _D), lambda i: (i, 0)),
        pl.BlockSpec((_D, _D), lambda i: (0, 0)),
        pl.BlockSpec((_D, _D), lambda i: (0, 0)),
        pl.BlockSpec((1, _D), lambda i: (0, 0)),
    ],
    out_specs=[pl.BlockSpec((_BLK, _D), lambda i: (i, 0))] * 2,
    out_shape=[jax.ShapeDtypeStruct((_N, _D), jnp.float32)] * 2,
)


def _mean_prelu(p_ref, cnt_ref, r_ref, a_ref):
  cnt = cnt_ref[0, :, 0:1] + cnt_ref[1, :, 0:1]
  scale = 1.0 / jnp.maximum(cnt, 1.0)
  h = (p_ref[0] + p_ref[1]) * scale + r_ref[...]
  a = a_ref[...]
  return jnp.where(h >= 0, h, a * h)


def _mid_body(p_ref, cnt_ref, r_ref, a_ref, wl_ref, wr_ref, b_ref, y_ref, r1_ref):
  h = _mean_prelu(p_ref, cnt_ref, r_ref, a_ref)
  y_ref[...] = jnp.dot(h, wl_ref[...], preferred_element_type=jnp.float32)
  r1_ref[...] = jnp.dot(h, wr_ref[...], preferred_element_type=jnp.float32) + b_ref[...]


_mid = pl.pallas_call(
    _mid_body,
    grid=(_N // _BLK,),
    in_specs=[
        pl.BlockSpec((_NC, _BLK, _D), lambda i: (0, i, 0)),
        pl.BlockSpec((_NC, _BLK, _D), lambda i: (0, i, 0)),
        pl.BlockSpec((_BLK, _D), lambda i: (i, 0)),
        pl.BlockSpec((1, _D), lambda i: (0, 0)),
        pl.BlockSpec((_D, _D), lambda i: (0, 0)),
        pl.BlockSpec((_D, _D), lambda i: (0, 0)),
        pl.BlockSpec((1, _D), lambda i: (0, 0)),
    ],
    out_specs=[pl.BlockSpec((_BLK, _D), lambda i: (i, 0))] * 2,
    out_shape=[jax.ShapeDtypeStruct((_N, _D), jnp.float32)] * 2,
)


def _fin_body(p_ref, cnt_ref, r_ref, a_ref, o_ref):
  o_ref[...] = _mean_prelu(p_ref, cnt_ref, r_ref, a_ref)


_fin = pl.pallas_call(
    _fin_body,
    grid=(_N // _BLK,),
    in_specs=[
        pl.BlockSpec((_NC, _BLK, _D), lambda i: (0, i, 0)),
        pl.BlockSpec((_NC, _BLK, _D), lambda i: (0, i, 0)),
        pl.BlockSpec((_BLK, _D), lambda i: (i, 0)),
        pl.BlockSpec((1, _D), lambda i: (0, 0)),
    ],
    out_specs=pl.BlockSpec((_BLK, _D), lambda i: (i, 0)),
    out_shape=jax.ShapeDtypeStruct((_N, _D), jnp.float32),
)


def kernel(x, edge_index, W_l0, b_l0, W_r0, a0, W_l1, b_l1, W_r1, a1):
  src = edge_index[0].astype(jnp.int32)
  dst = edge_index[1].astype(jnp.int32)
  src_p = jnp.concatenate([src, jnp.zeros((_EPAD - _E,), jnp.int32)])
  dst_p = jnp.concatenate([dst, jnp.full((_EPAD - _E,), _N, jnp.int32)])

  cnt = _make_cnt()(dst_p)
  y0, r0 = _pre(x, W_l0.T, W_r0.T, b_l0.reshape(1, _D))
  p0 = _make_agg()(y0, src_p, dst_p)
  y1, r1 = _mid(p0, cnt, r0, a0.reshape(1, _D),
                W_l1.T, W_r1.T, b_l1.reshape(1, _D))
  p1 = _make_agg()(y1, src_p, dst_p)
  return _fin(p1, cnt, r1, a1.reshape(1, _D))


# speedup vs baseline: 3.4815x; 1.1808x over previous
import functools

import jax
import jax.numpy as jnp
from jax import lax
from jax.experimental import pallas as pl
from jax.experimental.pallas import tpu as pltpu
from jax.experimental.pallas import tpu_sc as plsc

_N = 10000
_E = 320000
_D = 128
_NC = 2
_NS = 16
_NW = _NC * _NS
_L = 16
_NPAD = 10112
_RPT = _NPAD // _NS
_EPAD = 327680
_EPT = _EPAD // _NW
_CH = 128
_NCHUNK = _EPT // _CH
_BLK = 2000

_mesh = functools.lru_cache(maxsize=None)(
    lambda: plsc.VectorSubcoreMesh(core_axis_name="c", subcore_axis_name="s",
                                   num_cores=_NC, num_subcores=_NS))


def _fill(ref, value):
  v16 = jnp.full((_L,), value, jnp.float32)

  def row(r, carry):
    for cc in range(_D // _L):
      ref[r, pl.ds(cc * _L, _L)] = v16
    return carry

  lax.fori_loop(0, _CH, row, 0)


def _clear_stripe(acc, rbase, zeros_v):
  for k in range(4):
    pltpu.sync_copy(zeros_v, acc.at[pl.ds(rbase + k * _CH, _CH)])
  pltpu.sync_copy(zeros_v.at[pl.ds(0, _RPT - 4 * _CH)],
                  acc.at[pl.ds(rbase + 4 * _CH, _RPT - 4 * _CH)])


_NBUF = 4


@functools.lru_cache(maxsize=None)
def _make_agg():
  NG = _NCHUNK // 2

  def body(y_hbm, src_hbm, dst_hbm, part_hbm, acc, srcv, dstv, rows,
           isems, gsems, ssems):
    c = lax.axis_index("c")
    s = lax.axis_index("s")
    wid = s * _NC + c
    cbase = wid * _NCHUNK
    rbase = s * _RPT

    def issue_idx(j, b):
      pltpu.async_copy(src_hbm.at[cbase + j], srcv[b], isems[b])
      pltpu.async_copy(dst_hbm.at[cbase + j], dstv[b], isems[b])

    def wait_idx(j, b):
      pltpu.make_async_copy(src_hbm.at[cbase + j], srcv[b], isems[b]).wait()
      pltpu.make_async_copy(dst_hbm.at[cbase + j], dstv[b], isems[b]).wait()

    _fill(rows[0], 0.0)
    _clear_stripe(acc, rbase, rows[0])
    plsc.subcore_barrier()

    issue_idx(0, 0)
    issue_idx(1, 1)
    for b in (0, 1):
      wait_idx(b, b)
      pltpu.async_copy(y_hbm.at[srcv[b]], rows[b], gsems[b])

    def stage(g, pf_gather):
      for b in (0, 1):
        j = 2 * g + b
        pltpu.make_async_copy(y_hbm.at[srcv[b]], rows[b], gsems[b]).wait()
        pltpu.async_copy(rows[b], acc.at[dstv[b]], ssems[b], add=True)
        if pf_gather:
          pltpu.async_copy(src_hbm.at[cbase + j + 2], srcv[b], isems[b])
      for b in (0, 1):
        j = 2 * (g + 1) + b
        pltpu.make_async_copy(rows[b], acc.at[dstv[b]], ssems[b]).wait()
        if pf_gather:
          pltpu.make_async_copy(src_hbm.at[cbase + j], srcv[b],
                                isems[b]).wait()
          pltpu.async_copy(dst_hbm.at[cbase + j], dstv[b], isems[b])
          pltpu.async_copy(y_hbm.at[srcv[b]], rows[b], gsems[b])
          pltpu.make_async_copy(dst_hbm.at[cbase + j], dstv[b],
                                isems[b]).wait()

    lax.fori_loop(0, NG - 1, lambda g, cy: (stage(g, True), cy)[1], 0)
    stage(NG - 1, False)

    plsc.subcore_barrier()
    pltpu.sync_copy(acc.at[pl.ds(rbase, _RPT)],
                    part_hbm.at[c, pl.ds(rbase, _RPT)])

  return pl.kernel(
      body,
      out_type=jax.ShapeDtypeStruct((_NC, _NPAD, _D), jnp.float32),
      mesh=_mesh(),
      scratch_types=[
          pltpu.VMEM_SHARED((_NPAD, _D), jnp.float32),
          [pltpu.VMEM((_CH,), jnp.int32) for _ in range(2)],
          [pltpu.VMEM((_CH,), jnp.int32) for _ in range(2)],
          [pltpu.VMEM((_CH, _D), jnp.float32) for _ in range(2)],
          [pltpu.SemaphoreType.DMA for _ in range(2)],
          [pltpu.SemaphoreType.DMA for _ in range(2)],
          [pltpu.SemaphoreType.DMA for _ in range(2)],
      ])


@functools.lru_cache(maxsize=None)
def _make_cnt():
  def body(dst_hbm, cnt_hbm, acc, dst_all, ones_v, ssems):
    c = lax.axis_index("c")
    s = lax.axis_index("s")
    wid = s * _NC + c
    cbase = wid * _NCHUNK
    rbase = s * _RPT

    _fill(ones_v, 0.0)
    _clear_stripe(acc, rbase, ones_v)
    _fill(ones_v, 1.0)
    pltpu.sync_copy(dst_hbm.at[pl.ds(cbase, _NCHUNK)], dst_all)
    plsc.subcore_barrier()

    for b in range(_NBUF):
      pltpu.async_copy(ones_v, acc.at[dst_all.at[b]], ssems[b], add=True)

    def stage(g, prefetch):
      for b in range(_NBUF):
        j = g * _NBUF + b
        pltpu.make_async_copy(ones_v, acc.at[dst_all.at[j]], ssems[b]).wait()
        if prefetch:
          pltpu.async_copy(ones_v, acc.at[dst_all.at[j + _NBUF]], ssems[b],
                           add=True)

    lax.fori_loop(0, _NCHUNK // _NBUF - 1,
                  lambda g, carry: (stage(g, True), carry)[1], 0)
    stage(_NCHUNK // _NBUF - 1, False)

    plsc.subcore_barrier()
    pltpu.sync_copy(acc.at[pl.ds(rbase, _RPT)],
                    cnt_hbm.at[c, pl.ds(rbase, _RPT)])

  return pl.kernel(
      body,
      out_type=jax.ShapeDtypeStruct((_NC, _NPAD, _D), jnp.float32),
      mesh=_mesh(),
      scratch_types=[
          pltpu.VMEM_SHARED((_NPAD, _D), jnp.float32),
          pltpu.VMEM((_NCHUNK, _CH), jnp.int32),
          pltpu.VMEM((_CH, _D), jnp.float32),
          [pltpu.SemaphoreType.DMA for _ in range(_NBUF)],
      ])


def _pre_body(h_ref, wl_ref, wr_ref, b_ref, y_ref, r_ref):
  h = h_ref[...]
  y_ref[...] = jnp.dot(h, wl_ref[...], preferred_element_type=jnp.float32)
  r_ref[...] = jnp.dot(h, wr_ref[...], preferred_element_type=jnp.float32) + b_ref[...]


_pre = pl.pallas_call(
    _pre_body,
    grid=(_N // _BLK,),
    in_specs=[
        pl.BlockSpec((_BLK, _D), lambda i: (i, 0)),
        pl.BlockSpec((_D, _D), lambda i: (0, 0)),
        pl.BlockSpec((_D, _D), lambda i: (0, 0)),
        pl.BlockSpec((1, _D), lambda i: (0, 0)),
    ],
    out_specs=[pl.BlockSpec((_BLK, _D), lambda i: (i, 0))] * 2,
    out_shape=[jax.ShapeDtypeStruct((_N, _D), jnp.float32)] * 2,
)


def _mean_prelu(p_ref, cnt_ref, r_ref, a_ref):
  cnt = cnt_ref[0, :, 0:1] + cnt_ref[1, :, 0:1]
  scale = 1.0 / jnp.maximum(cnt, 1.0)
  h = (p_ref[0] + p_ref[1]) * scale + r_ref[...]
  a = a_ref[...]
  return jnp.where(h >= 0, h, a * h)


def _mid_body(p_ref, cnt_ref, r_ref, a_ref, wl_ref, wr_ref, b_ref, y_ref, r1_ref):
  h = _mean_prelu(p_ref, cnt_ref, r_ref, a_ref)
  y_ref[...] = jnp.dot(h, wl_ref[...], preferred_element_type=jnp.float32)
  r1_ref[...] = jnp.dot(h, wr_ref[...], preferred_element_type=jnp.float32) + b_ref[...]


_mid = pl.pallas_call(
    _mid_body,
    grid=(_N // _BLK,),
    in_specs=[
        pl.BlockSpec((_NC, _BLK, _D), lambda i: (0, i, 0)),
        pl.BlockSpec((_NC, _BLK, _D), lambda i: (0, i, 0)),
        pl.BlockSpec((_BLK, _D), lambda i: (i, 0)),
        pl.BlockSpec((1, _D), lambda i: (0, 0)),
        pl.BlockSpec((_D, _D), lambda i: (0, 0)),
        pl.BlockSpec((_D, _D), lambda i: (0, 0)),
        pl.BlockSpec((1, _D), lambda i: (0, 0)),
    ],
    out_specs=[pl.BlockSpec((_BLK, _D), lambda i: (i, 0))] * 2,
    out_shape=[jax.ShapeDtypeStruct((_N, _D), jnp.float32)] * 2,
)


def _fin_body(p_ref, cnt_ref, r_ref, a_ref, o_ref):
  o_ref[...] = _mean_prelu(p_ref, cnt_ref, r_ref, a_ref)


_fin = pl.pallas_call(
    _fin_body,
    grid=(_N // _BLK,),
    in_specs=[
        pl.BlockSpec((_NC, _BLK, _D), lambda i: (0, i, 0)),
        pl.BlockSpec((_NC, _BLK, _D), lambda i: (0, i, 0)),
        pl.BlockSpec((_BLK, _D), lambda i: (i, 0)),
        pl.BlockSpec((1, _D), lambda i: (0, 0)),
    ],
    out_specs=pl.BlockSpec((_BLK, _D), lambda i: (i, 0)),
    out_shape=jax.ShapeDtypeStruct((_N, _D), jnp.float32),
)


def kernel(x, edge_index, W_l0, b_l0, W_r0, a0, W_l1, b_l1, W_r1, a1):
  src = edge_index[0].astype(jnp.int32)
  dst = edge_index[1].astype(jnp.int32)
  src_p = jnp.concatenate(
      [src, jnp.zeros((_EPAD - _E,), jnp.int32)]).reshape(_EPAD // _CH, _CH)
  dst_p = jnp.concatenate(
      [dst, jnp.full((_EPAD - _E,), _N, jnp.int32)]).reshape(_EPAD // _CH, _CH)

  cnt = _make_cnt()(dst_p)
  y0, r0 = _pre(x, W_l0.T, W_r0.T, b_l0.reshape(1, _D))
  p0 = _make_agg()(y0, src_p, dst_p)
  y1, r1 = _mid(p0, cnt, r0, a0.reshape(1, _D),
                W_l1.T, W_r1.T, b_l1.reshape(1, _D))
  p1 = _make_agg()(y1, src_p, dst_p)
  return _fin(p1, cnt, r1, a1.reshape(1, _D))


# speedup vs baseline: 3.5134x; 1.0092x over previous
import functools

import jax
import jax.numpy as jnp
from jax import lax
from jax.experimental import pallas as pl
from jax.experimental.pallas import tpu as pltpu
from jax.experimental.pallas import tpu_sc as plsc

_N = 10000
_E = 320000
_D = 128
_NC = 2
_NS = 16
_NW = _NC * _NS
_L = 16
_NPAD = 10112
_RPT = _NPAD // _NS
_EPAD = 327680
_EPT = _EPAD // _NW
_CH = 128
_NCHUNK = _EPT // _CH
_BLK = 2000

_mesh = functools.lru_cache(maxsize=None)(
    lambda: plsc.VectorSubcoreMesh(core_axis_name="c", subcore_axis_name="s",
                                   num_cores=_NC, num_subcores=_NS))


def _fill(ref, value):
  v16 = jnp.full((_L,), value, jnp.float32)

  def row(r, carry):
    for cc in range(_D // _L):
      ref[r, pl.ds(cc * _L, _L)] = v16
    return carry

  lax.fori_loop(0, _CH, row, 0)


def _clear_stripe(acc, rbase, zeros_v):
  for k in range(4):
    pltpu.sync_copy(zeros_v, acc.at[pl.ds(rbase + k * _CH, _CH)])
  pltpu.sync_copy(zeros_v.at[pl.ds(0, _RPT - 4 * _CH)],
                  acc.at[pl.ds(rbase + 4 * _CH, _RPT - 4 * _CH)])


_NBUF = 4
_K0 = 40
_K1 = 2 * _NCHUNK - _K0


@functools.lru_cache(maxsize=None)
def _make_agg():
  def body(y_hbm, src_hbm, dst_hbm, part_hbm, acc, srcv, dstv, rows,
           isems, gsems, ssems):
    c = lax.axis_index("c")
    s = lax.axis_index("s")
    rbase = s * _RPT
    K = jnp.where(c == 0, _K0, _K1)
    cbase = jnp.where(c == 0, s * _K0, _NS * _K0 + s * _K1)
    NG = K // 2

    def issue_idx(j, b):
      pltpu.async_copy(src_hbm.at[cbase + j], srcv[b], isems[b])
      pltpu.async_copy(dst_hbm.at[cbase + j], dstv[b], isems[b])

    def wait_idx(j, b):
      pltpu.make_async_copy(src_hbm.at[cbase + j], srcv[b], isems[b]).wait()
      pltpu.make_async_copy(dst_hbm.at[cbase + j], dstv[b], isems[b]).wait()

    _fill(rows[0], 0.0)
    _clear_stripe(acc, rbase, rows[0])
    plsc.subcore_barrier()

    issue_idx(0, 0)
    issue_idx(1, 1)
    for b in (0, 1):
      wait_idx(b, b)
      pltpu.async_copy(y_hbm.at[srcv[b]], rows[b], gsems[b])

    def stage(g, pf_gather):
      for b in (0, 1):
        j = 2 * g + b
        pltpu.make_async_copy(y_hbm.at[srcv[b]], rows[b], gsems[b]).wait()
        pltpu.async_copy(rows[b], acc.at[dstv[b]], ssems[b], add=True)
        if pf_gather:
          pltpu.async_copy(src_hbm.at[cbase + j + 2], srcv[b], isems[b])
      for b in (0, 1):
        j = 2 * (g + 1) + b
        pltpu.make_async_copy(rows[b], acc.at[dstv[b]], ssems[b]).wait()
        if pf_gather:
          pltpu.make_async_copy(src_hbm.at[cbase + j], srcv[b],
                                isems[b]).wait()
          pltpu.async_copy(dst_hbm.at[cbase + j], dstv[b], isems[b])
          pltpu.async_copy(y_hbm.at[srcv[b]], rows[b], gsems[b])
          pltpu.make_async_copy(dst_hbm.at[cbase + j], dstv[b],
                                isems[b]).wait()

    lax.fori_loop(0, NG - 1, lambda g, cy: (stage(g, True), cy)[1], 0)
    stage(NG - 1, False)

    plsc.subcore_barrier()
    pltpu.sync_copy(acc.at[pl.ds(rbase, _RPT)],
                    part_hbm.at[c, pl.ds(rbase, _RPT)])

  return pl.kernel(
      body,
      out_type=jax.ShapeDtypeStruct((_NC, _NPAD, _D), jnp.float32),
      mesh=_mesh(),
      scratch_types=[
          pltpu.VMEM_SHARED((_NPAD, _D), jnp.float32),
          [pltpu.VMEM((_CH,), jnp.int32) for _ in range(2)],
          [pltpu.VMEM((_CH,), jnp.int32) for _ in range(2)],
          [pltpu.VMEM((_CH, _D), jnp.float32) for _ in range(2)],
          [pltpu.SemaphoreType.DMA for _ in range(2)],
          [pltpu.SemaphoreType.DMA for _ in range(2)],
          [pltpu.SemaphoreType.DMA for _ in range(2)],
      ])


@functools.lru_cache(maxsize=None)
def _make_cnt():
  def body(dst_hbm, cnt_hbm, acc, dst_all, ones_v, ssems):
    c = lax.axis_index("c")
    s = lax.axis_index("s")
    wid = s * _NC + c
    cbase = wid * _NCHUNK
    rbase = s * _RPT

    _fill(ones_v, 0.0)
    _clear_stripe(acc, rbase, ones_v)
    _fill(ones_v, 1.0)
    pltpu.sync_copy(dst_hbm.at[pl.ds(cbase, _NCHUNK)], dst_all)
    plsc.subcore_barrier()

    for b in range(_NBUF):
      pltpu.async_copy(ones_v, acc.at[dst_all.at[b]], ssems[b], add=True)

    def stage(g, prefetch):
      for b in range(_NBUF):
        j = g * _NBUF + b
        pltpu.make_async_copy(ones_v, acc.at[dst_all.at[j]], ssems[b]).wait()
        if prefetch:
          pltpu.async_copy(ones_v, acc.at[dst_all.at[j + _NBUF]], ssems[b],
                           add=True)

    lax.fori_loop(0, _NCHUNK // _NBUF - 1,
                  lambda g, carry: (stage(g, True), carry)[1], 0)
    stage(_NCHUNK // _NBUF - 1, False)

    plsc.subcore_barrier()
    pltpu.sync_copy(acc.at[pl.ds(rbase, _RPT)],
                    cnt_hbm.at[c, pl.ds(rbase, _RPT)])

  return pl.kernel(
      body,
      out_type=jax.ShapeDtypeStruct((_NC, _NPAD, _D), jnp.float32),
      mesh=_mesh(),
      scratch_types=[
          pltpu.VMEM_SHARED((_NPAD, _D), jnp.float32),
          pltpu.VMEM((_NCHUNK, _CH), jnp.int32),
          pltpu.VMEM((_CH, _D), jnp.float32),
          [pltpu.SemaphoreType.DMA for _ in range(_NBUF)],
      ])


def _pre_body(h_ref, wl_ref, wr_ref, b_ref, y_ref, r_ref):
  h = h_ref[...]
  y_ref[...] = jnp.dot(h, wl_ref[...], preferred_element_type=jnp.float32)
  r_ref[...] = jnp.dot(h, wr_ref[...], preferred_element_type=jnp.float32) + b_ref[...]


_pre = pl.pallas_call(
    _pre_body,
    grid=(_N // _BLK,),
    in_specs=[
        pl.BlockSpec((_BLK, _D), lambda i: (i, 0)),
        pl.BlockSpec((_D, _D), lambda i: (0, 0)),
        pl.BlockSpec((_D, _D), lambda i: (0, 0)),
        pl.BlockSpec((1, _D), lambda i: (0, 0)),
    ],
    out_specs=[pl.BlockSpec((_BLK, _D), lambda i: (i, 0))] * 2,
    out_shape=[jax.ShapeDtypeStruct((_N, _D), jnp.float32)] * 2,
)


def _mean_prelu(p_ref, cnt_ref, r_ref, a_ref):
  cnt = cnt_ref[0, :, 0:1] + cnt_ref[1, :, 0:1]
  scale = 1.0 / jnp.maximum(cnt, 1.0)
  h = (p_ref[0] + p_ref[1]) * scale + r_ref[...]
  a = a_ref[...]
  return jnp.where(h >= 0, h, a * h)


def _mid_body(p_ref, cnt_ref, r_ref, a_ref, wl_ref, wr_ref, b_ref, y_ref, r1_ref):
  h = _mean_prelu(p_ref, cnt_ref, r_ref, a_ref)
  y_ref[...] = jnp.dot(h, wl_ref[...], preferred_element_type=jnp.float32)
  r1_ref[...] = jnp.dot(h, wr_ref[...], preferred_element_type=jnp.float32) + b_ref[...]


_mid = pl.pallas_call(
    _mid_body,
    grid=(_N // _BLK,),
    in_specs=[
        pl.BlockSpec((_NC, _BLK, _D), lambda i: (0, i, 0)),
        pl.BlockSpec((_NC, _BLK, _D), lambda i: (0, i, 0)),
        pl.BlockSpec((_BLK, _D), lambda i: (i, 0)),
        pl.BlockSpec((1, _D), lambda i: (0, 0)),
        pl.BlockSpec((_D, _D), lambda i: (0, 0)),
        pl.BlockSpec((_D, _D), lambda i: (0, 0)),
        pl.BlockSpec((1, _D), lambda i: (0, 0)),
    ],
    out_specs=[pl.BlockSpec((_BLK, _D), lambda i: (i, 0))] * 2,
    out_shape=[jax.ShapeDtypeStruct((_N, _D), jnp.float32)] * 2,
)


def _fin_body(p_ref, cnt_ref, r_ref, a_ref, o_ref):
  o_ref[...] = _mean_prelu(p_ref, cnt_ref, r_ref, a_ref)


_fin = pl.pallas_call(
    _fin_body,
    grid=(_N // _BLK,),
    in_specs=[
        pl.BlockSpec((_NC, _BLK, _D), lambda i: (0, i, 0)),
        pl.BlockSpec((_NC, _BLK, _D), lambda i: (0, i, 0)),
        pl.BlockSpec((_BLK, _D), lambda i: (i, 0)),
        pl.BlockSpec((1, _D), lambda i: (0, 0)),
    ],
    out_specs=pl.BlockSpec((_BLK, _D), lambda i: (i, 0)),
    out_shape=jax.ShapeDtypeStruct((_N, _D), jnp.float32),
)


def kernel(x, edge_index, W_l0, b_l0, W_r0, a0, W_l1, b_l1, W_r1, a1):
  src = edge_index[0].astype(jnp.int32)
  dst = edge_index[1].astype(jnp.int32)
  src_p = jnp.concatenate(
      [src, jnp.zeros((_EPAD - _E,), jnp.int32)]).reshape(_EPAD // _CH, _CH)
  dst_p = jnp.concatenate(
      [dst, jnp.full((_EPAD - _E,), _N, jnp.int32)]).reshape(_EPAD // _CH, _CH)

  cnt = _make_cnt()(dst_p)
  y0, r0 = _pre(x, W_l0.T, W_r0.T, b_l0.reshape(1, _D))
  p0 = _make_agg()(y0, src_p, dst_p)
  y1, r1 = _mid(p0, cnt, r0, a0.reshape(1, _D),
                W_l1.T, W_r1.T, b_l1.reshape(1, _D))
  p1 = _make_agg()(y1, src_p, dst_p)
  return _fin(p1, cnt, r1, a1.reshape(1, _D))


# speedup vs baseline: 3.9505x; 1.1244x over previous
import functools

import jax
import jax.numpy as jnp
from jax import lax
from jax.experimental import pallas as pl
from jax.experimental.pallas import tpu as pltpu
from jax.experimental.pallas import tpu_sc as plsc

_N = 10000
_E = 320000
_D = 128
_NC = 2
_NS = 16
_NW = _NC * _NS
_L = 16
_NPAD = 10112
_RPT = _NPAD // _NS
_EPAD = 327680
_EPT = _EPAD // _NW
_CH = 128
_NCHUNK = _EPT // _CH
_BLK = 2000

_mesh = functools.lru_cache(maxsize=None)(
    lambda: plsc.VectorSubcoreMesh(core_axis_name="c", subcore_axis_name="s",
                                   num_cores=_NC, num_subcores=_NS))


def _fill(ref, value):
  v16 = jnp.full((_L,), value, jnp.float32)

  def row(r, carry):
    for cc in range(_D // _L):
      ref[r, pl.ds(cc * _L, _L)] = v16
    return carry

  lax.fori_loop(0, _CH, row, 0)


def _clear_stripe(acc, rbase, zeros_v):
  for k in range(4):
    pltpu.sync_copy(zeros_v, acc.at[pl.ds(rbase + k * _CH, _CH)])
  pltpu.sync_copy(zeros_v.at[pl.ds(0, _RPT - 4 * _CH)],
                  acc.at[pl.ds(rbase + 4 * _CH, _RPT - 4 * _CH)])


_NBUF = 4
_K0 = 120
_K1 = 2 * _NCHUNK - _K0


@functools.lru_cache(maxsize=None)
def _make_agg():
  def body(y_hbm, src_hbm, dst_hbm, part_hbm, acc, srcv, dstv, rows,
           isems, gsems, ssems):
    c = lax.axis_index("c")
    s = lax.axis_index("s")
    rbase = s * _RPT
    K = jnp.where(c == 0, _K0, _K1)
    cbase = jnp.where(c == 0, s * _K0, _NS * _K0 + s * _K1)
    NG = K // 2

    def issue_idx(j, b):
      pltpu.async_copy(src_hbm.at[cbase + j], srcv[b], isems[b])
      pltpu.async_copy(dst_hbm.at[cbase + j], dstv[b], isems[b])

    def wait_idx(j, b):
      pltpu.make_async_copy(src_hbm.at[cbase + j], srcv[b], isems[b]).wait()
      pltpu.make_async_copy(dst_hbm.at[cbase + j], dstv[b], isems[b]).wait()

    _fill(rows[0], 0.0)
    _clear_stripe(acc, rbase, rows[0])
    plsc.subcore_barrier()

    issue_idx(0, 0)
    issue_idx(1, 1)
    for b in (0, 1):
      wait_idx(b, b)
      pltpu.async_copy(y_hbm.at[srcv[b]], rows[b], gsems[b])

    def stage(g, pf_gather):
      for b in (0, 1):
        j = 2 * g + b
        pltpu.make_async_copy(y_hbm.at[srcv[b]], rows[b], gsems[b]).wait()
        pltpu.async_copy(rows[b], acc.at[dstv[b]], ssems[b], add=True)
        if pf_gather:
          pltpu.async_copy(src_hbm.at[cbase + j + 2], srcv[b], isems[b])
      for b in (0, 1):
        j = 2 * (g + 1) + b
        pltpu.make_async_copy(rows[b], acc.at[dstv[b]], ssems[b]).wait()
        if pf_gather:
          pltpu.make_async_copy(src_hbm.at[cbase + j], srcv[b],
                                isems[b]).wait()
          pltpu.async_copy(dst_hbm.at[cbase + j], dstv[b], isems[b])
          pltpu.async_copy(y_hbm.at[srcv[b]], rows[b], gsems[b])
          pltpu.make_async_copy(dst_hbm.at[cbase + j], dstv[b],
                                isems[b]).wait()

    lax.fori_loop(0, NG - 1, lambda g, cy: (stage(g, True), cy)[1], 0)
    stage(NG - 1, False)

    plsc.subcore_barrier()
    pltpu.sync_copy(acc.at[pl.ds(rbase, _RPT)],
                    part_hbm.at[c, pl.ds(rbase, _RPT)])

  return pl.kernel(
      body,
      out_type=jax.ShapeDtypeStruct((_NC, _NPAD, _D), jnp.float32),
      mesh=_mesh(),
      scratch_types=[
          pltpu.VMEM_SHARED((_NPAD, _D), jnp.float32),
          [pltpu.VMEM((_CH,), jnp.int32) for _ in range(2)],
          [pltpu.VMEM((_CH,), jnp.int32) for _ in range(2)],
          [pltpu.VMEM((_CH, _D), jnp.float32) for _ in range(2)],
          [pltpu.SemaphoreType.DMA for _ in range(2)],
          [pltpu.SemaphoreType.DMA for _ in range(2)],
          [pltpu.SemaphoreType.DMA for _ in range(2)],
      ])


@functools.lru_cache(maxsize=None)
def _make_cnt():
  def body(dst_hbm, cnt_hbm, acc, dst_all, ones_v, ssems):
    c = lax.axis_index("c")
    s = lax.axis_index("s")
    wid = s * _NC + c
    cbase = wid * _NCHUNK
    rbase = s * _RPT

    _fill(ones_v, 0.0)
    _clear_stripe(acc, rbase, ones_v)
    _fill(ones_v, 1.0)
    pltpu.sync_copy(dst_hbm.at[pl.ds(cbase, _NCHUNK)], dst_all)
    plsc.subcore_barrier()

    for b in range(_NBUF):
      pltpu.async_copy(ones_v, acc.at[dst_all.at[b]], ssems[b], add=True)

    def stage(g, prefetch):
      for b in range(_NBUF):
        j = g * _NBUF + b
        pltpu.make_async_copy(ones_v, acc.at[dst_all.at[j]], ssems[b]).wait()
        if prefetch:
          pltpu.async_copy(ones_v, acc.at[dst_all.at[j + _NBUF]], ssems[b],
                           add=True)

    lax.fori_loop(0, _NCHUNK // _NBUF - 1,
                  lambda g, carry: (stage(g, True), carry)[1], 0)
    stage(_NCHUNK // _NBUF - 1, False)

    plsc.subcore_barrier()
    pltpu.sync_copy(acc.at[pl.ds(rbase, _RPT)],
                    cnt_hbm.at[c, pl.ds(rbase, _RPT)])

  return pl.kernel(
      body,
      out_type=jax.ShapeDtypeStruct((_NC, _NPAD, _D), jnp.float32),
      mesh=_mesh(),
      scratch_types=[
          pltpu.VMEM_SHARED((_NPAD, _D), jnp.float32),
          pltpu.VMEM((_NCHUNK, _CH), jnp.int32),
          pltpu.VMEM((_CH, _D), jnp.float32),
          [pltpu.SemaphoreType.DMA for _ in range(_NBUF)],
      ])


def _pre_body(h_ref, wl_ref, wr_ref, b_ref, y_ref, r_ref):
  h = h_ref[...]
  y_ref[...] = jnp.dot(h, wl_ref[...], preferred_element_type=jnp.float32)
  r_ref[...] = jnp.dot(h, wr_ref[...], preferred_element_type=jnp.float32) + b_ref[...]


_pre = pl.pallas_call(
    _pre_body,
    grid=(_N // _BLK,),
    in_specs=[
        pl.BlockSpec((_BLK, _D), lambda i: (i, 0)),
        pl.BlockSpec((_D, _D), lambda i: (0, 0)),
        pl.BlockSpec((_D, _D), lambda i: (0, 0)),
        pl.BlockSpec((1, _D), lambda i: (0, 0)),
    ],
    out_specs=[pl.BlockSpec((_BLK, _D), lambda i: (i, 0))] * 2,
    out_shape=[jax.ShapeDtypeStruct((_N, _D), jnp.float32)] * 2,
)


def _mean_prelu(p_ref, cnt_ref, r_ref, a_ref):
  cnt = cnt_ref[0, :, 0:1] + cnt_ref[1, :, 0:1]
  scale = 1.0 / jnp.maximum(cnt, 1.0)
  h = (p_ref[0] + p_ref[1]) * scale + r_ref[...]
  a = a_ref[...]
  return jnp.where(h >= 0, h, a * h)


def _mid_body(p_ref, cnt_ref, r_ref, a_ref, wl_ref, wr_ref, b_ref, y_ref, r1_ref):
  h = _mean_prelu(p_ref, cnt_ref, r_ref, a_ref)
  y_ref[...] = jnp.dot(h, wl_ref[...], preferred_element_type=jnp.float32)
  r1_ref[...] = jnp.dot(h, wr_ref[...], preferred_element_type=jnp.float32) + b_ref[...]


_mid = pl.pallas_call(
    _mid_body,
    grid=(_N // _BLK,),
    in_specs=[
        pl.BlockSpec((_NC, _BLK, _D), lambda i: (0, i, 0)),
        pl.BlockSpec((_NC, _BLK, _D), lambda i: (0, i, 0)),
        pl.BlockSpec((_BLK, _D), lambda i: (i, 0)),
        pl.BlockSpec((1, _D), lambda i: (0, 0)),
        pl.BlockSpec((_D, _D), lambda i: (0, 0)),
        pl.BlockSpec((_D, _D), lambda i: (0, 0)),
        pl.BlockSpec((1, _D), lambda i: (0, 0)),
    ],
    out_specs=[pl.BlockSpec((_BLK, _D), lambda i: (i, 0))] * 2,
    out_shape=[jax.ShapeDtypeStruct((_N, _D), jnp.float32)] * 2,
)


def _fin_body(p_ref, cnt_ref, r_ref, a_ref, o_ref):
  o_ref[...] = _mean_prelu(p_ref, cnt_ref, r_ref, a_ref)


_fin = pl.pallas_call(
    _fin_body,
    grid=(_N // _BLK,),
    in_specs=[
        pl.BlockSpec((_NC, _BLK, _D), lambda i: (0, i, 0)),
        pl.BlockSpec((_NC, _BLK, _D), lambda i: (0, i, 0)),
        pl.BlockSpec((_BLK, _D), lambda i: (i, 0)),
        pl.BlockSpec((1, _D), lambda i: (0, 0)),
    ],
    out_specs=pl.BlockSpec((_BLK, _D), lambda i: (i, 0)),
    out_shape=jax.ShapeDtypeStruct((_N, _D), jnp.float32),
)


def kernel(x, edge_index, W_l0, b_l0, W_r0, a0, W_l1, b_l1, W_r1, a1):
  src = edge_index[0].astype(jnp.int32)
  dst = edge_index[1].astype(jnp.int32)
  src_p = jnp.concatenate(
      [src, jnp.zeros((_EPAD - _E,), jnp.int32)]).reshape(_EPAD // _CH, _CH)
  dst_p = jnp.concatenate(
      [dst, jnp.full((_EPAD - _E,), _N, jnp.int32)]).reshape(_EPAD // _CH, _CH)

  cnt = _make_cnt()(dst_p)
  y0, r0 = _pre(x, W_l0.T, W_r0.T, b_l0.reshape(1, _D))
  p0 = _make_agg()(y0, src_p, dst_p)
  y1, r1 = _mid(p0, cnt, r0, a0.reshape(1, _D),
                W_l1.T, W_r1.T, b_l1.reshape(1, _D))
  p1 = _make_agg()(y1, src_p, dst_p)
  return _fin(p1, cnt, r1, a1.reshape(1, _D))
